# Initial kernel scaffold; baseline (speedup 1.0000x reference)
import jax
import jax.numpy as jnp
from jax.experimental import pallas as pl


def kernel(src, q_w, q_b, k_w, k_b, v_w, v_b, o_w, o_b, gate_w, w1, w2, norm1_w, norm2_w):
    raise NotImplementedError("write your pallas kernel here")


# speedup vs baseline: 1.2330x; 1.2330x over previous
import functools
import jax
import jax.numpy as jnp
from jax import lax
from jax.experimental import pallas as pl
from jax.experimental.pallas import tpu as pltpu
from jax.experimental.pallas import tpu_sc as plsc

B = 2
S = 2048
D = 1024
H = 16
KVH = 4
DH = D // H
REP = H // KVH
DFF = 2048
E = 64
K = 2
N = B * S
CAP = int(1.25 * N * K / E)
NSLOT = E * CAP
NPAD = NSLOT + 8
QB = 512
MB = 256
SCALE = 1.0 / (DH ** 0.5)


def _proj_body(x_ref, w_ref, b_ref, o_ref):
    o_ref[...] = (
        jnp.dot(x_ref[...], w_ref[...], preferred_element_type=jnp.float32)
        + b_ref[...]
    )


def _attn_body(q_ref, k_ref, v_ref, o_ref):
    q = q_ref[0, 0]
    k = k_ref[0, 0]
    v = v_ref[0, 0]
    s = lax.dot_general(q, k, (((1,), (1,)), ((), ())),
                        preferred_element_type=jnp.float32) * SCALE
    m = jnp.max(s, axis=1, keepdims=True)
    p = jnp.exp(s - m)
    l = jnp.sum(p, axis=1, keepdims=True)
    o_ref[0, 0] = jnp.dot(p / l, v, preferred_element_type=jnp.float32)


def _oproj_body(a_ref, w_ref, b_ref, src_ref, n1_ref, gw_ref, x1_ref, p_ref):
    y = (
        jnp.dot(a_ref[...], w_ref[...], preferred_element_type=jnp.float32)
        + b_ref[...]
        + src_ref[...]
    )
    var = jnp.mean(y * y, axis=1, keepdims=True)
    x1 = n1_ref[...] * (y * lax.rsqrt(var + 1e-6))
    x1_ref[...] = x1
    p_ref[...] = jax.nn.sigmoid(
        jnp.dot(x1, gw_ref[...], preferred_element_type=jnp.float32)
    )


def _route_body(p_ref, s0_ref, s1_ref, g0_ref, g1_ref, r0_ref, r1_ref,
                cnt_ref, carry):
    pid = pl.program_id(0)

    @pl.when(pid == 0)
    def _():
        carry[...] = jnp.zeros_like(carry)

    probs = p_ref[...]
    lanes = lax.broadcasted_iota(jnp.int32, (MB, E), 1)
    m1 = jnp.max(probs, axis=1, keepdims=True)
    a1 = jnp.min(jnp.where(probs == m1, lanes, E), axis=1, keepdims=True)
    pm = jnp.where(lanes == a1, -jnp.inf, probs)
    m2 = jnp.max(pm, axis=1, keepdims=True)
    a2 = jnp.min(jnp.where(pm == m2, lanes, E), axis=1, keepdims=True)
    oh0 = (lanes == a1).astype(jnp.float32)
    oh1 = (lanes == a2).astype(jnp.float32)
    ohs = oh0 + oh1
    ii = lax.broadcasted_iota(jnp.int32, (MB, MB), 0)
    jj = lax.broadcasted_iota(jnp.int32, (MB, MB), 1)
    tri = (jj < ii).astype(jnp.float32)
    cumex = (
        jnp.dot(tri, ohs, preferred_element_type=jnp.float32) + carry[...]
    )
    rank0 = jnp.sum(cumex * oh0, axis=1, keepdims=True)
    rank1 = jnp.sum(cumex * oh1, axis=1, keepdims=True)
    carry[...] = carry[...] + jnp.sum(ohs, axis=0, keepdims=True)
    v0 = rank0 < CAP
    v1 = rank1 < CAP
    sid0 = a1 * CAP + rank0.astype(jnp.int32)
    sid1 = a2 * CAP + rank1.astype(jnp.int32)
    s0_ref[...] = jnp.where(v0, sid0, NSLOT)
    s1_ref[...] = jnp.where(v1, sid1, NSLOT)
    g0_ref[...] = jnp.where(v0, sid0, 0)
    g1_ref[...] = jnp.where(v1, sid1, 0)
    ssum = m1 + m2 + 1e-6
    r0_ref[...] = jnp.where(v0, m1 / ssum, 0.0)
    r1_ref[...] = jnp.where(v1, m2 / ssum, 0.0)
    cnt_ref[...] = jnp.minimum(carry[...], float(CAP)).astype(jnp.int32)


def _ffn_body(cnt_ref, x_ref, w1_ref, w2_ref, o_ref):
    e = pl.program_id(0)
    c = cnt_ref[0, e]
    rows = lax.broadcasted_iota(jnp.int32, (CAP, 1), 0)
    x = jnp.where(rows < c, x_ref[...], 0.0)
    h = jnp.dot(x, w1_ref[0], preferred_element_type=jnp.float32)
    h = 0.5 * h * (1.0 + lax.erf(h * (2.0 ** -0.5)))
    o_ref[...] = jnp.dot(h, w2_ref[0], preferred_element_type=jnp.float32)


def _combine_body(x1_ref, g0_ref, g1_ref, r0_ref, r1_ref, n2_ref, o_ref):
    y = x1_ref[...] + r0_ref[...] * g0_ref[...] + r1_ref[...] * g1_ref[...]
    var = jnp.mean(y * y, axis=1, keepdims=True)
    o_ref[...] = n2_ref[...] * (y * lax.rsqrt(var + 1e-6))


_RPW = 8
_CH = 32


def _disp_body(xf_hbm, sidx_hbm, padded_hbm, idx_v, rows_v, sem):
    info = plsc.get_sparse_core_info()
    nc = info.num_cores
    wid = lax.axis_index("s") * nc + lax.axis_index("c")
    base = wid * _RPW
    pltpu.sync_copy(sidx_hbm.at[pl.ds(base, _RPW)], idx_v)
    for jl in range(_RPW):
        j = base + jl
        tok = lax.rem(j, 128) * _CH
        pltpu.sync_copy(xf_hbm.at[pl.ds(tok, _CH)], rows_v)
        pltpu.async_copy(rows_v, padded_hbm.at[idx_v.at[jl]], sem).wait()


def _gath_body(outp_hbm, gidx_hbm, gout_hbm, idx_v, rows_v, sem):
    info = plsc.get_sparse_core_info()
    nc = info.num_cores
    wid = lax.axis_index("s") * nc + lax.axis_index("c")
    base = wid * _RPW
    pltpu.sync_copy(gidx_hbm.at[pl.ds(base, _RPW)], idx_v)
    for jl in range(_RPW):
        j = base + jl
        pltpu.async_copy(outp_hbm.at[idx_v.at[jl]], rows_v, sem).wait()
        pltpu.sync_copy(rows_v, gout_hbm.at[pl.ds(j * _CH, _CH)])


@jax.jit
def kernel(src, q_w, q_b, k_w, k_b, v_w, v_b, o_w, o_b, gate_w, w1, w2,
           norm1_w, norm2_w):
    f32 = jnp.float32
    xf = src.reshape(N, D)

    wcat = jnp.concatenate([q_w, k_w, v_w], axis=0).T
    bcat = jnp.concatenate([q_b, k_b, v_b]).reshape(1, -1)
    qkv = pl.pallas_call(
        _proj_body,
        grid=(N // QB,),
        in_specs=[
            pl.BlockSpec((QB, D), lambda i: (i, 0)),
            pl.BlockSpec((D, D + 2 * KVH * DH), lambda i: (0, 0)),
            pl.BlockSpec((1, D + 2 * KVH * DH), lambda i: (0, 0)),
        ],
        out_specs=pl.BlockSpec((QB, D + 2 * KVH * DH), lambda i: (i, 0)),
        out_shape=jax.ShapeDtypeStruct((N, D + 2 * KVH * DH), f32),
    )(xf, wcat, bcat)
    qkvt = qkv.reshape(B, S, H + 2 * KVH, DH).transpose(0, 2, 1, 3)

    attn = pl.pallas_call(
        _attn_body,
        grid=(B, H, S // QB),
        in_specs=[
            pl.BlockSpec((1, 1, QB, DH), lambda b, h, i: (b, h, i, 0)),
            pl.BlockSpec((1, 1, S, DH),
                         lambda b, h, i: (b, H + h // REP, 0, 0)),
            pl.BlockSpec((1, 1, S, DH),
                         lambda b, h, i: (b, H + KVH + h // REP, 0, 0)),
        ],
        out_specs=pl.BlockSpec((1, 1, QB, DH), lambda b, h, i: (b, h, i, 0)),
        out_shape=jax.ShapeDtypeStruct((B, H, S, DH), f32),
    )(qkvt, qkvt, qkvt)
    attn = attn.transpose(0, 2, 1, 3).reshape(N, D)

    x1, probs = pl.pallas_call(
        _oproj_body,
        grid=(N // MB,),
        in_specs=[
            pl.BlockSpec((MB, D), lambda i: (i, 0)),
            pl.BlockSpec((D, D), lambda i: (0, 0)),
            pl.BlockSpec((1, D), lambda i: (0, 0)),
            pl.BlockSpec((MB, D), lambda i: (i, 0)),
            pl.BlockSpec((1, D), lambda i: (0, 0)),
            pl.BlockSpec((D, E), lambda i: (0, 0)),
        ],
        out_specs=[
            pl.BlockSpec((MB, D), lambda i: (i, 0)),
            pl.BlockSpec((MB, E), lambda i: (i, 0)),
        ],
        out_shape=[
            jax.ShapeDtypeStruct((N, D), f32),
            jax.ShapeDtypeStruct((N, E), f32),
        ],
    )(attn, o_w.T, o_b.reshape(1, D), xf, norm1_w.reshape(1, D), gate_w.T)

    i32 = jnp.int32
    s0, s1, g0i, g1i, r0, r1, cnt = pl.pallas_call(
        _route_body,
        grid=(N // MB,),
        in_specs=[pl.BlockSpec((MB, E), lambda i: (i, 0))],
        out_specs=[
            pl.BlockSpec((MB, 1), lambda i: (i, 0)),
            pl.BlockSpec((MB, 1), lambda i: (i, 0)),
            pl.BlockSpec((MB, 1), lambda i: (i, 0)),
            pl.BlockSpec((MB, 1), lambda i: (i, 0)),
            pl.BlockSpec((MB, 1), lambda i: (i, 0)),
            pl.BlockSpec((MB, 1), lambda i: (i, 0)),
            pl.BlockSpec((1, E), lambda i: (0, 0)),
        ],
        out_shape=[
            jax.ShapeDtypeStruct((N, 1), i32),
            jax.ShapeDtypeStruct((N, 1), i32),
            jax.ShapeDtypeStruct((N, 1), i32),
            jax.ShapeDtypeStruct((N, 1), i32),
            jax.ShapeDtypeStruct((N, 1), f32),
            jax.ShapeDtypeStruct((N, 1), f32),
            jax.ShapeDtypeStruct((1, E), i32),
        ],
        scratch_shapes=[pltpu.VMEM((1, E), f32)],
    )(probs)

    sidx = jnp.concatenate([s0, s1], axis=0).reshape(N * K // _CH, _CH)
    gidx = jnp.concatenate([g0i, g1i], axis=0).reshape(N * K // _CH, _CH)

    mesh = plsc.VectorSubcoreMesh(core_axis_name="c", subcore_axis_name="s")

    padded = pl.kernel(
        _disp_body,
        out_type=jax.ShapeDtypeStruct((NPAD, D), f32),
        mesh=mesh,
        scratch_types=[
            pltpu.VMEM((_RPW, _CH), i32),
            pltpu.VMEM((_CH, D), f32),
            pltpu.SemaphoreType.DMA,
        ],
    )(x1, sidx)

    outp = pl.pallas_call(
        _ffn_body,
        grid=(E,),
        in_specs=[
            pl.BlockSpec(memory_space=pltpu.SMEM),
            pl.BlockSpec((CAP, D), lambda e: (e, 0)),
            pl.BlockSpec((1, D, DFF), lambda e: (e, 0, 0)),
            pl.BlockSpec((1, DFF, D), lambda e: (e, 0, 0)),
        ],
        out_specs=pl.BlockSpec((CAP, D), lambda e: (e, 0)),
        out_shape=jax.ShapeDtypeStruct((NSLOT, D), f32),
    )(cnt, padded, w1, w2)

    gout = pl.kernel(
        _gath_body,
        out_type=jax.ShapeDtypeStruct((N * K, D), f32),
        mesh=mesh,
        scratch_types=[
            pltpu.VMEM((_RPW, _CH), i32),
            pltpu.VMEM((_CH, D), f32),
            pltpu.SemaphoreType.DMA,
        ],
    )(outp, gidx)

    out = pl.pallas_call(
        _combine_body,
        grid=(N // MB,),
        in_specs=[
            pl.BlockSpec((MB, D), lambda i: (i, 0)),
            pl.BlockSpec((MB, D), lambda i: (i, 0)),
            pl.BlockSpec((MB, D), lambda i: (i + N // MB, 0)),
            pl.BlockSpec((MB, 1), lambda i: (i, 0)),
            pl.BlockSpec((MB, 1), lambda i: (i, 0)),
            pl.BlockSpec((1, D), lambda i: (0, 0)),
        ],
        out_specs=pl.BlockSpec((MB, D), lambda i: (i, 0)),
        out_shape=jax.ShapeDtypeStruct((N, D), f32),
    )(x1, gout, gout, r0, r1, norm2_w.reshape(1, D))

    return out.reshape(B, S, D)


# speedup vs baseline: 1.6073x; 1.3036x over previous
import functools
import jax
import jax.numpy as jnp
from jax import lax
from jax.experimental import pallas as pl
from jax.experimental.pallas import tpu as pltpu
from jax.experimental.pallas import tpu_sc as plsc

B = 2
S = 2048
D = 1024
H = 16
KVH = 4
DH = D // H
REP = H // KVH
DFF = 2048
E = 64
K = 2
N = B * S
CAP = int(1.25 * N * K / E)
NSLOT = E * CAP
NPAD = NSLOT + 8
QB = 512
MB = 256
SCALE = 1.0 / (DH ** 0.5)


def _proj_body(x_ref, w_ref, b_ref, q_ref, kv_ref):
    y = (
        jnp.dot(x_ref[...], w_ref[...], preferred_element_type=jnp.float32)
        + b_ref[...]
    )
    q_ref[...] = y[:, :D]
    kv_ref[...] = y[:, D:]


def _attn_body(q_ref, kv_ref, o_ref):
    kv = kv_ref[...]
    for h in range(H):
        q = q_ref[:, h * DH:(h + 1) * DH]
        kh = h // REP
        k = kv[:, kh * DH:(kh + 1) * DH]
        v = kv[:, KVH * DH + kh * DH:KVH * DH + (kh + 1) * DH]
        s = lax.dot_general(q, k, (((1,), (1,)), ((), ())),
                            preferred_element_type=jnp.float32) * SCALE
        m = jnp.max(s, axis=1, keepdims=True)
        p = jnp.exp(s - m)
        l = jnp.sum(p, axis=1, keepdims=True)
        o_ref[:, h * DH:(h + 1) * DH] = jnp.dot(
            p / l, v, preferred_element_type=jnp.float32)


def _oproj_body(a_ref, w_ref, b_ref, src_ref, n1_ref, gw_ref, x1_ref, p_ref):
    y = (
        jnp.dot(a_ref[...], w_ref[...], preferred_element_type=jnp.float32)
        + b_ref[...]
        + src_ref[...]
    )
    var = jnp.mean(y * y, axis=1, keepdims=True)
    x1 = n1_ref[...] * (y * lax.rsqrt(var + 1e-6))
    x1_ref[...] = x1
    p_ref[...] = jax.nn.sigmoid(
        jnp.dot(x1, gw_ref[...], preferred_element_type=jnp.float32)
    )


def _route_body(p_ref, s0_ref, s1_ref, g0_ref, g1_ref, r0_ref, r1_ref,
                cnt_ref, carry):
    pid = pl.program_id(0)

    @pl.when(pid == 0)
    def _():
        carry[...] = jnp.zeros_like(carry)

    probs = p_ref[...]
    lanes = lax.broadcasted_iota(jnp.int32, (MB, E), 1)
    m1 = jnp.max(probs, axis=1, keepdims=True)
    a1 = jnp.min(jnp.where(probs == m1, lanes, E), axis=1, keepdims=True)
    pm = jnp.where(lanes == a1, -jnp.inf, probs)
    m2 = jnp.max(pm, axis=1, keepdims=True)
    a2 = jnp.min(jnp.where(pm == m2, lanes, E), axis=1, keepdims=True)
    oh0 = (lanes == a1).astype(jnp.float32)
    oh1 = (lanes == a2).astype(jnp.float32)
    ohs = oh0 + oh1
    ii = lax.broadcasted_iota(jnp.int32, (MB, MB), 0)
    jj = lax.broadcasted_iota(jnp.int32, (MB, MB), 1)
    tri = (jj < ii).astype(jnp.float32)
    cumex = (
        jnp.dot(tri, ohs, preferred_element_type=jnp.float32) + carry[...]
    )
    rank0 = jnp.sum(cumex * oh0, axis=1, keepdims=True)
    rank1 = jnp.sum(cumex * oh1, axis=1, keepdims=True)
    carry[...] = carry[...] + jnp.sum(ohs, axis=0, keepdims=True)
    v0 = rank0 < CAP
    v1 = rank1 < CAP
    sid0 = a1 * CAP + rank0.astype(jnp.int32)
    sid1 = a2 * CAP + rank1.astype(jnp.int32)
    s0_ref[...] = jnp.where(v0, sid0, NSLOT)
    s1_ref[...] = jnp.where(v1, sid1, NSLOT)
    g0_ref[...] = jnp.where(v0, sid0, 0)
    g1_ref[...] = jnp.where(v1, sid1, 0)
    ssum = m1 + m2 + 1e-6
    r0_ref[...] = jnp.where(v0, m1 / ssum, 0.0)
    r1_ref[...] = jnp.where(v1, m2 / ssum, 0.0)
    cnt_ref[...] = jnp.minimum(carry[...], float(CAP)).astype(jnp.int32)


def _ffn_body(cnt_ref, x_ref, w1_ref, w2_ref, o_ref):
    e = pl.program_id(0)
    c = cnt_ref[0, e]
    rows = lax.broadcasted_iota(jnp.int32, (CAP, 1), 0)
    x = jnp.where(rows < c, x_ref[...], 0.0)
    h = jnp.dot(x, w1_ref[0], preferred_element_type=jnp.float32)
    h = 0.5 * h * (1.0 + lax.erf(h * (2.0 ** -0.5)))
    o_ref[...] = jnp.dot(h, w2_ref[0], preferred_element_type=jnp.float32)


def _combine_body(x1_ref, g0_ref, g1_ref, r0_ref, r1_ref, n2_ref, o_ref):
    y = x1_ref[...] + r0_ref[...] * g0_ref[...] + r1_ref[...] * g1_ref[...]
    var = jnp.mean(y * y, axis=1, keepdims=True)
    o_ref[...] = n2_ref[...] * (y * lax.rsqrt(var + 1e-6))


_RPW = 8
_CH = 32


def _disp_body(xf_hbm, sidx_hbm, padded_hbm, idx_v, rows_v, sem):
    info = plsc.get_sparse_core_info()
    nc = info.num_cores
    wid = lax.axis_index("s") * nc + lax.axis_index("c")
    base = wid * _RPW
    pltpu.sync_copy(sidx_hbm.at[pl.ds(base, _RPW)], idx_v)
    for jl in range(_RPW):
        j = base + jl
        tok = lax.rem(j, 128) * _CH
        pltpu.sync_copy(xf_hbm.at[pl.ds(tok, _CH)], rows_v)
        pltpu.async_copy(rows_v, padded_hbm.at[idx_v.at[jl]], sem).wait()


def _gath_body(outp_hbm, gidx_hbm, gout_hbm, idx_v, rows_v, sem):
    info = plsc.get_sparse_core_info()
    nc = info.num_cores
    wid = lax.axis_index("s") * nc + lax.axis_index("c")
    base = wid * _RPW
    pltpu.sync_copy(gidx_hbm.at[pl.ds(base, _RPW)], idx_v)
    for jl in range(_RPW):
        j = base + jl
        pltpu.async_copy(outp_hbm.at[idx_v.at[jl]], rows_v, sem).wait()
        pltpu.sync_copy(rows_v, gout_hbm.at[pl.ds(j * _CH, _CH)])


@jax.jit
def kernel(src, q_w, q_b, k_w, k_b, v_w, v_b, o_w, o_b, gate_w, w1, w2,
           norm1_w, norm2_w):
    f32 = jnp.float32
    xf = src.reshape(N, D)

    wcat = jnp.concatenate([q_w, k_w, v_w], axis=0).T
    bcat = jnp.concatenate([q_b, k_b, v_b]).reshape(1, -1)
    q, kvc = pl.pallas_call(
        _proj_body,
        grid=(N // QB,),
        in_specs=[
            pl.BlockSpec((QB, D), lambda i: (i, 0)),
            pl.BlockSpec((D, D + 2 * KVH * DH), lambda i: (0, 0)),
            pl.BlockSpec((1, D + 2 * KVH * DH), lambda i: (0, 0)),
        ],
        out_specs=[
            pl.BlockSpec((QB, D), lambda i: (i, 0)),
            pl.BlockSpec((QB, 2 * KVH * DH), lambda i: (i, 0)),
        ],
        out_shape=[
            jax.ShapeDtypeStruct((N, D), f32),
            jax.ShapeDtypeStruct((N, 2 * KVH * DH), f32),
        ],
    )(xf, wcat, bcat)

    attn = pl.pallas_call(
        _attn_body,
        grid=(B, S // QB),
        in_specs=[
            pl.BlockSpec((QB, D), lambda b, i: (b * (S // QB) + i, 0)),
            pl.BlockSpec((S, 2 * KVH * DH), lambda b, i: (b, 0)),
        ],
        out_specs=pl.BlockSpec((QB, D), lambda b, i: (b * (S // QB) + i, 0)),
        out_shape=jax.ShapeDtypeStruct((N, D), f32),
    )(q, kvc)

    x1, probs = pl.pallas_call(
        _oproj_body,
        grid=(N // MB,),
        in_specs=[
            pl.BlockSpec((MB, D), lambda i: (i, 0)),
            pl.BlockSpec((D, D), lambda i: (0, 0)),
            pl.BlockSpec((1, D), lambda i: (0, 0)),
            pl.BlockSpec((MB, D), lambda i: (i, 0)),
            pl.BlockSpec((1, D), lambda i: (0, 0)),
            pl.BlockSpec((D, E), lambda i: (0, 0)),
        ],
        out_specs=[
            pl.BlockSpec((MB, D), lambda i: (i, 0)),
            pl.BlockSpec((MB, E), lambda i: (i, 0)),
        ],
        out_shape=[
            jax.ShapeDtypeStruct((N, D), f32),
            jax.ShapeDtypeStruct((N, E), f32),
        ],
    )(attn, o_w.T, o_b.reshape(1, D), xf, norm1_w.reshape(1, D), gate_w.T)

    i32 = jnp.int32
    s0, s1, g0i, g1i, r0, r1, cnt = pl.pallas_call(
        _route_body,
        grid=(N // MB,),
        in_specs=[pl.BlockSpec((MB, E), lambda i: (i, 0))],
        out_specs=[
            pl.BlockSpec((MB, 1), lambda i: (i, 0)),
            pl.BlockSpec((MB, 1), lambda i: (i, 0)),
            pl.BlockSpec((MB, 1), lambda i: (i, 0)),
            pl.BlockSpec((MB, 1), lambda i: (i, 0)),
            pl.BlockSpec((MB, 1), lambda i: (i, 0)),
            pl.BlockSpec((MB, 1), lambda i: (i, 0)),
            pl.BlockSpec((1, E), lambda i: (0, 0)),
        ],
        out_shape=[
            jax.ShapeDtypeStruct((N, 1), i32),
            jax.ShapeDtypeStruct((N, 1), i32),
            jax.ShapeDtypeStruct((N, 1), i32),
            jax.ShapeDtypeStruct((N, 1), i32),
            jax.ShapeDtypeStruct((N, 1), f32),
            jax.ShapeDtypeStruct((N, 1), f32),
            jax.ShapeDtypeStruct((1, E), i32),
        ],
        scratch_shapes=[pltpu.VMEM((1, E), f32)],
    )(probs)

    sidx = jnp.concatenate([s0, s1], axis=0).reshape(N * K // _CH, _CH)
    gidx = jnp.concatenate([g0i, g1i], axis=0).reshape(N * K // _CH, _CH)

    mesh = plsc.VectorSubcoreMesh(core_axis_name="c", subcore_axis_name="s")

    padded = pl.kernel(
        _disp_body,
        out_type=jax.ShapeDtypeStruct((NPAD, D), f32),
        mesh=mesh,
        scratch_types=[
            pltpu.VMEM((_RPW, _CH), i32),
            pltpu.VMEM((_CH, D), f32),
            pltpu.SemaphoreType.DMA,
        ],
    )(x1, sidx)

    outp = pl.pallas_call(
        _ffn_body,
        grid=(E,),
        in_specs=[
            pl.BlockSpec(memory_space=pltpu.SMEM),
            pl.BlockSpec((CAP, D), lambda e: (e, 0)),
            pl.BlockSpec((1, D, DFF), lambda e: (e, 0, 0)),
            pl.BlockSpec((1, DFF, D), lambda e: (e, 0, 0)),
        ],
        out_specs=pl.BlockSpec((CAP, D), lambda e: (e, 0)),
        out_shape=jax.ShapeDtypeStruct((NSLOT, D), f32),
    )(cnt, padded, w1, w2)

    gout = pl.kernel(
        _gath_body,
        out_type=jax.ShapeDtypeStruct((N * K, D), f32),
        mesh=mesh,
        scratch_types=[
            pltpu.VMEM((_RPW, _CH), i32),
            pltpu.VMEM((_CH, D), f32),
            pltpu.SemaphoreType.DMA,
        ],
    )(outp, gidx)

    out = pl.pallas_call(
        _combine_body,
        grid=(N // MB,),
        in_specs=[
            pl.BlockSpec((MB, D), lambda i: (i, 0)),
            pl.BlockSpec((MB, D), lambda i: (i, 0)),
            pl.BlockSpec((MB, D), lambda i: (i + N // MB, 0)),
            pl.BlockSpec((MB, 1), lambda i: (i, 0)),
            pl.BlockSpec((MB, 1), lambda i: (i, 0)),
            pl.BlockSpec((1, D), lambda i: (0, 0)),
        ],
        out_specs=pl.BlockSpec((MB, D), lambda i: (i, 0)),
        out_shape=jax.ShapeDtypeStruct((N, D), f32),
    )(x1, gout, gout, r0, r1, norm2_w.reshape(1, D))

    return out.reshape(B, S, D)


# speedup vs baseline: 1.8175x; 1.1308x over previous
import functools
import jax
import jax.numpy as jnp
from jax import lax
from jax.experimental import pallas as pl
from jax.experimental.pallas import tpu as pltpu
from jax.experimental.pallas import tpu_sc as plsc

B = 2
S = 2048
D = 1024
H = 16
KVH = 4
DH = D // H
REP = H // KVH
DFF = 2048
E = 64
K = 2
N = B * S
CAP = int(1.25 * N * K / E)
NSLOT = E * CAP
NPAD = NSLOT + 8
QB = 512
AQB = 256
MB = 256
SCALE = 1.0 / (DH ** 0.5)


def _proj_body(x_ref, w_ref, b_ref, q_ref, kv_ref):
    y = (
        jnp.dot(x_ref[...], w_ref[...], preferred_element_type=jnp.float32)
        + b_ref[...]
    )
    q_ref[...] = y[:, :D] * SCALE
    kv_ref[...] = y[:, D:]


def _attn_body(q_ref, kv_ref, o_ref):
    kv = kv_ref[...]
    for h in range(H):
        q = q_ref[:, h * DH:(h + 1) * DH]
        kh = h // REP
        k = kv[:, kh * DH:(kh + 1) * DH]
        v = kv[:, KVH * DH + kh * DH:KVH * DH + (kh + 1) * DH]
        s = lax.dot_general(q, k, (((1,), (1,)), ((), ())),
                            preferred_element_type=jnp.float32)
        m = jnp.max(s, axis=1, keepdims=True)
        p = jnp.exp(s - m)
        l = jnp.sum(p, axis=1, keepdims=True)
        o = jnp.dot(p, v, preferred_element_type=jnp.float32)
        o_ref[:, h * DH:(h + 1) * DH] = o / l


def _oproj_route_body(a_ref, w_ref, b_ref, src_ref, n1_ref, gw_ref,
                      x1_ref, s0_ref, s1_ref, g0_ref, g1_ref, r0_ref, r1_ref,
                      cnt_ref, carry):
    y = (
        jnp.dot(a_ref[...], w_ref[...], preferred_element_type=jnp.float32)
        + b_ref[...]
        + src_ref[...]
    )
    var = jnp.mean(y * y, axis=1, keepdims=True)
    x1 = n1_ref[...] * (y * lax.rsqrt(var + 1e-6))
    x1_ref[...] = x1
    probs = jax.nn.sigmoid(
        jnp.dot(x1, gw_ref[...], preferred_element_type=jnp.float32)
    )

    pid = pl.program_id(0)

    @pl.when(pid == 0)
    def _():
        carry[...] = jnp.zeros_like(carry)

    lanes = lax.broadcasted_iota(jnp.int32, (MB, E), 1)
    m1 = jnp.max(probs, axis=1, keepdims=True)
    a1 = jnp.min(jnp.where(probs == m1, lanes, E), axis=1, keepdims=True)
    pm = jnp.where(lanes == a1, -jnp.inf, probs)
    m2 = jnp.max(pm, axis=1, keepdims=True)
    a2 = jnp.min(jnp.where(pm == m2, lanes, E), axis=1, keepdims=True)
    oh0 = (lanes == a1).astype(jnp.float32)
    oh1 = (lanes == a2).astype(jnp.float32)
    ohs = oh0 + oh1
    ii = lax.broadcasted_iota(jnp.int32, (MB, MB), 0)
    jj = lax.broadcasted_iota(jnp.int32, (MB, MB), 1)
    tri = (jj < ii).astype(jnp.float32)
    cumex = (
        jnp.dot(tri, ohs, preferred_element_type=jnp.float32) + carry[...]
    )
    rank0 = jnp.sum(cumex * oh0, axis=1, keepdims=True)
    rank1 = jnp.sum(cumex * oh1, axis=1, keepdims=True)
    carry[...] = carry[...] + jnp.sum(ohs, axis=0, keepdims=True)
    v0 = rank0 < CAP
    v1 = rank1 < CAP
    sid0 = a1 * CAP + rank0.astype(jnp.int32)
    sid1 = a2 * CAP + rank1.astype(jnp.int32)
    s0_ref[...] = jnp.where(v0, sid0, NSLOT)
    s1_ref[...] = jnp.where(v1, sid1, NSLOT)
    g0_ref[...] = jnp.where(v0, sid0, 0)
    g1_ref[...] = jnp.where(v1, sid1, 0)
    ssum = m1 + m2 + 1e-6
    r0_ref[...] = jnp.where(v0, m1 / ssum, 0.0)
    r1_ref[...] = jnp.where(v1, m2 / ssum, 0.0)
    cnt_ref[...] = jnp.minimum(carry[...], float(CAP)).astype(jnp.int32)


def _ffn_body(cnt_ref, x_ref, w1_ref, w2_ref, o_ref):
    e = pl.program_id(0)
    c = cnt_ref[0, e]
    rows = lax.broadcasted_iota(jnp.int32, (CAP, 1), 0)
    x = jnp.where(rows < c, x_ref[...], 0.0)
    h = jnp.dot(x, w1_ref[0], preferred_element_type=jnp.float32)
    h = 0.5 * h * (1.0 + lax.erf(h * (2.0 ** -0.5)))
    o_ref[...] = jnp.dot(h, w2_ref[0], preferred_element_type=jnp.float32)


def _combine_body(x1_ref, g0_ref, g1_ref, r0_ref, r1_ref, n2_ref, o_ref):
    y = x1_ref[...] + r0_ref[...] * g0_ref[...] + r1_ref[...] * g1_ref[...]
    var = jnp.mean(y * y, axis=1, keepdims=True)
    o_ref[...] = n2_ref[...] * (y * lax.rsqrt(var + 1e-6))


_RPW = 8
_CH = 32


def _disp_body(xf_hbm, sidx_hbm, padded_hbm, idx_v, rows_v, sem):
    info = plsc.get_sparse_core_info()
    nc = info.num_cores
    wid = lax.axis_index("s") * nc + lax.axis_index("c")
    base = wid * _RPW
    pltpu.sync_copy(sidx_hbm.at[pl.ds(base, _RPW)], idx_v)
    for jl in range(_RPW):
        j = base + jl
        tok = lax.rem(j, 128) * _CH
        pltpu.sync_copy(xf_hbm.at[pl.ds(tok, _CH)], rows_v)
        pltpu.async_copy(rows_v, padded_hbm.at[idx_v.at[jl]], sem).wait()


def _gath_body(outp_hbm, gidx_hbm, gout_hbm, idx_v, rows_v, sem):
    info = plsc.get_sparse_core_info()
    nc = info.num_cores
    wid = lax.axis_index("s") * nc + lax.axis_index("c")
    base = wid * _RPW
    pltpu.sync_copy(gidx_hbm.at[pl.ds(base, _RPW)], idx_v)
    for jl in range(_RPW):
        j = base + jl
        pltpu.async_copy(outp_hbm.at[idx_v.at[jl]], rows_v, sem).wait()
        pltpu.sync_copy(rows_v, gout_hbm.at[pl.ds(j * _CH, _CH)])


@jax.jit
def kernel(src, q_w, q_b, k_w, k_b, v_w, v_b, o_w, o_b, gate_w, w1, w2,
           norm1_w, norm2_w):
    f32 = jnp.float32
    xf = src.reshape(N, D)

    wcat = jnp.concatenate([q_w, k_w, v_w], axis=0).T
    bcat = jnp.concatenate([q_b, k_b, v_b]).reshape(1, -1)
    q, kvc = pl.pallas_call(
        _proj_body,
        grid=(N // QB,),
        in_specs=[
            pl.BlockSpec((QB, D), lambda i: (i, 0)),
            pl.BlockSpec((D, D + 2 * KVH * DH), lambda i: (0, 0)),
            pl.BlockSpec((1, D + 2 * KVH * DH), lambda i: (0, 0)),
        ],
        out_specs=[
            pl.BlockSpec((QB, D), lambda i: (i, 0)),
            pl.BlockSpec((QB, 2 * KVH * DH), lambda i: (i, 0)),
        ],
        out_shape=[
            jax.ShapeDtypeStruct((N, D), f32),
            jax.ShapeDtypeStruct((N, 2 * KVH * DH), f32),
        ],
    )(xf, wcat, bcat)

    attn = pl.pallas_call(
        _attn_body,
        grid=(B, S // AQB),
        in_specs=[
            pl.BlockSpec((AQB, D), lambda b, i: (b * (S // AQB) + i, 0)),
            pl.BlockSpec((S, 2 * KVH * DH), lambda b, i: (b, 0)),
        ],
        out_specs=pl.BlockSpec((AQB, D), lambda b, i: (b * (S // AQB) + i, 0)),
        out_shape=jax.ShapeDtypeStruct((N, D), f32),
    )(q, kvc)

    i32 = jnp.int32
    x1, s0, s1, g0i, g1i, r0, r1, cnt = pl.pallas_call(
        _oproj_route_body,
        grid=(N // MB,),
        in_specs=[
            pl.BlockSpec((MB, D), lambda i: (i, 0)),
            pl.BlockSpec((D, D), lambda i: (0, 0)),
            pl.BlockSpec((1, D), lambda i: (0, 0)),
            pl.BlockSpec((MB, D), lambda i: (i, 0)),
            pl.BlockSpec((1, D), lambda i: (0, 0)),
            pl.BlockSpec((D, E), lambda i: (0, 0)),
        ],
        out_specs=[
            pl.BlockSpec((MB, D), lambda i: (i, 0)),
            pl.BlockSpec((MB, 1), lambda i: (i, 0)),
            pl.BlockSpec((MB, 1), lambda i: (i, 0)),
            pl.BlockSpec((MB, 1), lambda i: (i, 0)),
            pl.BlockSpec((MB, 1), lambda i: (i, 0)),
            pl.BlockSpec((MB, 1), lambda i: (i, 0)),
            pl.BlockSpec((MB, 1), lambda i: (i, 0)),
            pl.BlockSpec((1, E), lambda i: (0, 0)),
        ],
        out_shape=[
            jax.ShapeDtypeStruct((N, D), f32),
            jax.ShapeDtypeStruct((N, 1), i32),
            jax.ShapeDtypeStruct((N, 1), i32),
            jax.ShapeDtypeStruct((N, 1), i32),
            jax.ShapeDtypeStruct((N, 1), i32),
            jax.ShapeDtypeStruct((N, 1), f32),
            jax.ShapeDtypeStruct((N, 1), f32),
            jax.ShapeDtypeStruct((1, E), i32),
        ],
        scratch_shapes=[pltpu.VMEM((1, E), f32)],
    )(attn, o_w.T, o_b.reshape(1, D), xf, norm1_w.reshape(1, D), gate_w.T)

    sidx = jnp.concatenate([s0, s1], axis=0).reshape(N * K // _CH, _CH)
    gidx = jnp.concatenate([g0i, g1i], axis=0).reshape(N * K // _CH, _CH)

    mesh = plsc.VectorSubcoreMesh(core_axis_name="c", subcore_axis_name="s")

    padded = pl.kernel(
        _disp_body,
        out_type=jax.ShapeDtypeStruct((NPAD, D), f32),
        mesh=mesh,
        scratch_types=[
            pltpu.VMEM((_RPW, _CH), i32),
            pltpu.VMEM((_CH, D), f32),
            pltpu.SemaphoreType.DMA,
        ],
    )(x1, sidx)

    outp = pl.pallas_call(
        _ffn_body,
        grid=(E,),
        in_specs=[
            pl.BlockSpec(memory_space=pltpu.SMEM),
            pl.BlockSpec((CAP, D), lambda e: (e, 0)),
            pl.BlockSpec((1, D, DFF), lambda e: (e, 0, 0)),
            pl.BlockSpec((1, DFF, D), lambda e: (e, 0, 0)),
        ],
        out_specs=pl.BlockSpec((CAP, D), lambda e: (e, 0)),
        out_shape=jax.ShapeDtypeStruct((NSLOT, D), f32),
    )(cnt, padded, w1, w2)

    gout = pl.kernel(
        _gath_body,
        out_type=jax.ShapeDtypeStruct((N * K, D), f32),
        mesh=mesh,
        scratch_types=[
            pltpu.VMEM((_RPW, _CH), i32),
            pltpu.VMEM((_CH, D), f32),
            pltpu.SemaphoreType.DMA,
        ],
    )(outp, gidx)

    out = pl.pallas_call(
        _combine_body,
        grid=(N // MB,),
        in_specs=[
            pl.BlockSpec((MB, D), lambda i: (i, 0)),
            pl.BlockSpec((MB, D), lambda i: (i, 0)),
            pl.BlockSpec((MB, D), lambda i: (i + N // MB, 0)),
            pl.BlockSpec((MB, 1), lambda i: (i, 0)),
            pl.BlockSpec((MB, 1), lambda i: (i, 0)),
            pl.BlockSpec((1, D), lambda i: (0, 0)),
        ],
        out_specs=pl.BlockSpec((MB, D), lambda i: (i, 0)),
        out_shape=jax.ShapeDtypeStruct((N, D), f32),
    )(x1, gout, gout, r0, r1, norm2_w.reshape(1, D))

    return out.reshape(B, S, D)


# speedup vs baseline: 1.8429x; 1.0140x over previous
import functools
import jax
import jax.numpy as jnp
from jax import lax
from jax.experimental import pallas as pl
from jax.experimental.pallas import tpu as pltpu
from jax.experimental.pallas import tpu_sc as plsc

B = 2
S = 2048
D = 1024
H = 16
KVH = 4
DH = D // H
REP = H // KVH
DFF = 2048
E = 64
K = 2
N = B * S
CAP = int(1.25 * N * K / E)
NSLOT = E * CAP
NPAD = NSLOT + 8
QB = 512
AQB = 256
MB = 256
SCALE = 1.0 / (DH ** 0.5)


def _proj_body(x_ref, w_ref, b_ref, q_ref, kv_ref):
    y = (
        jnp.dot(x_ref[...], w_ref[...], preferred_element_type=jnp.float32)
        + b_ref[...]
    )
    q_ref[...] = y[:, :D] * SCALE
    kv_ref[...] = y[:, D:]


def _attn_body(q_ref, kv_ref, o_ref):
    kv = kv_ref[...]
    for h in range(H):
        q = q_ref[:, h * DH:(h + 1) * DH]
        kh = h // REP
        k = kv[:, kh * DH:(kh + 1) * DH]
        v = kv[:, KVH * DH + kh * DH:KVH * DH + (kh + 1) * DH]
        s = lax.dot_general(q, k, (((1,), (1,)), ((), ())),
                            preferred_element_type=jnp.float32)
        m = jnp.max(s, axis=1, keepdims=True)
        p = jnp.exp(s - m)
        l = jnp.sum(p, axis=1, keepdims=True)
        o = jnp.dot(p, v, preferred_element_type=jnp.float32)
        o_ref[:, h * DH:(h + 1) * DH] = o / l


def _oproj_route_body(a_ref, w_ref, b_ref, src_ref, n1_ref, gw_ref,
                      x1_ref, s0_ref, s1_ref, g0_ref, g1_ref, r0_ref, r1_ref,
                      cnt_ref, carry):
    y = (
        jnp.dot(a_ref[...], w_ref[...], preferred_element_type=jnp.float32)
        + b_ref[...]
        + src_ref[...]
    )
    var = jnp.mean(y * y, axis=1, keepdims=True)
    x1 = n1_ref[...] * (y * lax.rsqrt(var + 1e-6))
    x1_ref[...] = x1
    probs = jax.nn.sigmoid(
        jnp.dot(x1, gw_ref[...], preferred_element_type=jnp.float32)
    )

    pid = pl.program_id(0)

    @pl.when(pid == 0)
    def _():
        carry[...] = jnp.zeros_like(carry)

    lanes = lax.broadcasted_iota(jnp.int32, (MB, E), 1)
    m1 = jnp.max(probs, axis=1, keepdims=True)
    a1 = jnp.min(jnp.where(probs == m1, lanes, E), axis=1, keepdims=True)
    pm = jnp.where(lanes == a1, -jnp.inf, probs)
    m2 = jnp.max(pm, axis=1, keepdims=True)
    a2 = jnp.min(jnp.where(pm == m2, lanes, E), axis=1, keepdims=True)
    oh0 = (lanes == a1).astype(jnp.float32)
    oh1 = (lanes == a2).astype(jnp.float32)
    ohs = oh0 + oh1
    ii = lax.broadcasted_iota(jnp.int32, (MB, MB), 0)
    jj = lax.broadcasted_iota(jnp.int32, (MB, MB), 1)
    tri = (jj < ii).astype(jnp.float32)
    cumex = (
        jnp.dot(tri, ohs, preferred_element_type=jnp.float32) + carry[...]
    )
    rank0 = jnp.sum(cumex * oh0, axis=1, keepdims=True)
    rank1 = jnp.sum(cumex * oh1, axis=1, keepdims=True)
    carry[...] = carry[...] + jnp.sum(ohs, axis=0, keepdims=True)
    v0 = rank0 < CAP
    v1 = rank1 < CAP
    sid0 = a1 * CAP + rank0.astype(jnp.int32)
    sid1 = a2 * CAP + rank1.astype(jnp.int32)
    s0_ref[...] = jnp.where(v0, sid0, NSLOT)
    s1_ref[...] = jnp.where(v1, sid1, NSLOT)
    g0_ref[...] = jnp.where(v0, sid0, 0)
    g1_ref[...] = jnp.where(v1, sid1, 0)
    ssum = m1 + m2 + 1e-6
    r0_ref[...] = jnp.where(v0, m1 / ssum, 0.0)
    r1_ref[...] = jnp.where(v1, m2 / ssum, 0.0)
    cnt_ref[...] = jnp.minimum(carry[...], float(CAP)).astype(jnp.int32)


def _ffn_body(cnt_ref, x_ref, w1_ref, w2_ref, o_ref):
    e = pl.program_id(0)
    c = cnt_ref[0, e]
    rows = lax.broadcasted_iota(jnp.int32, (CAP, 1), 0)
    x = jnp.where(rows < c, x_ref[...], 0.0)
    h = jnp.dot(x, w1_ref[0], preferred_element_type=jnp.float32)
    h = 0.5 * h * (1.0 + lax.erf(h * (2.0 ** -0.5)))
    o_ref[...] = jnp.dot(h, w2_ref[0], preferred_element_type=jnp.float32)


def _combine_body(x1_ref, g0_ref, g1_ref, r0_ref, r1_ref, n2_ref, o_ref):
    y = x1_ref[...] + r0_ref[...] * g0_ref[...] + r1_ref[...] * g1_ref[...]
    var = jnp.mean(y * y, axis=1, keepdims=True)
    o_ref[...] = n2_ref[...] * (y * lax.rsqrt(var + 1e-6))


_RPW = 8
_CH = 32


def _disp_body(xf_hbm, sidx_hbm, padded_hbm, idx_v, rows0, rows1,
               lsem0, lsem1, ssem0, ssem1):
    info = plsc.get_sparse_core_info()
    nc = info.num_cores
    wid = lax.axis_index("s") * nc + lax.axis_index("c")
    base = wid * _RPW
    pltpu.sync_copy(sidx_hbm.at[pl.ds(base, _RPW)], idx_v)
    bufs = (rows0, rows1)
    lsems = (lsem0, lsem1)
    ssems = (ssem0, ssem1)

    def tok(jl):
        return lax.rem(base + jl, 128) * _CH

    loads = {}
    scats = {}
    loads[0] = pltpu.async_copy(xf_hbm.at[pl.ds(tok(0), _CH)], bufs[0],
                                lsems[0])
    for jl in range(_RPW):
        b = jl % 2
        if jl + 1 < _RPW:
            if jl - 1 >= 0:
                scats[jl - 1].wait()
            nb = (jl + 1) % 2
            loads[jl + 1] = pltpu.async_copy(
                xf_hbm.at[pl.ds(tok(jl + 1), _CH)], bufs[nb], lsems[nb])
        loads[jl].wait()
        scats[jl] = pltpu.async_copy(bufs[b], padded_hbm.at[idx_v.at[jl]],
                                     ssems[b])
    scats[_RPW - 2].wait()
    scats[_RPW - 1].wait()


def _gath_body(outp_hbm, gidx_hbm, gout_hbm, idx_v, rows0, rows1,
               lsem0, lsem1, ssem0, ssem1):
    info = plsc.get_sparse_core_info()
    nc = info.num_cores
    wid = lax.axis_index("s") * nc + lax.axis_index("c")
    base = wid * _RPW
    pltpu.sync_copy(gidx_hbm.at[pl.ds(base, _RPW)], idx_v)
    bufs = (rows0, rows1)
    lsems = (lsem0, lsem1)
    ssems = (ssem0, ssem1)
    loads = {}
    stores = {}
    loads[0] = pltpu.async_copy(outp_hbm.at[idx_v.at[0]], bufs[0], lsems[0])
    for jl in range(_RPW):
        b = jl % 2
        if jl + 1 < _RPW:
            if jl - 1 >= 0:
                stores[jl - 1].wait()
            nb = (jl + 1) % 2
            loads[jl + 1] = pltpu.async_copy(outp_hbm.at[idx_v.at[jl + 1]],
                                             bufs[nb], lsems[nb])
        loads[jl].wait()
        stores[jl] = pltpu.async_copy(
            bufs[b], gout_hbm.at[pl.ds((base + jl) * _CH, _CH)], ssems[b])
    stores[_RPW - 2].wait()
    stores[_RPW - 1].wait()


@jax.jit
def kernel(src, q_w, q_b, k_w, k_b, v_w, v_b, o_w, o_b, gate_w, w1, w2,
           norm1_w, norm2_w):
    f32 = jnp.float32
    xf = src.reshape(N, D)

    wcat = jnp.concatenate([q_w, k_w, v_w], axis=0).T
    bcat = jnp.concatenate([q_b, k_b, v_b]).reshape(1, -1)
    q, kvc = pl.pallas_call(
        _proj_body,
        grid=(N // QB,),
        in_specs=[
            pl.BlockSpec((QB, D), lambda i: (i, 0)),
            pl.BlockSpec((D, D + 2 * KVH * DH), lambda i: (0, 0)),
            pl.BlockSpec((1, D + 2 * KVH * DH), lambda i: (0, 0)),
        ],
        out_specs=[
            pl.BlockSpec((QB, D), lambda i: (i, 0)),
            pl.BlockSpec((QB, 2 * KVH * DH), lambda i: (i, 0)),
        ],
        out_shape=[
            jax.ShapeDtypeStruct((N, D), f32),
            jax.ShapeDtypeStruct((N, 2 * KVH * DH), f32),
        ],
    )(xf, wcat, bcat)

    attn = pl.pallas_call(
        _attn_body,
        grid=(B, S // AQB),
        in_specs=[
            pl.BlockSpec((AQB, D), lambda b, i: (b * (S // AQB) + i, 0)),
            pl.BlockSpec((S, 2 * KVH * DH), lambda b, i: (b, 0)),
        ],
        out_specs=pl.BlockSpec((AQB, D), lambda b, i: (b * (S // AQB) + i, 0)),
        out_shape=jax.ShapeDtypeStruct((N, D), f32),
    )(q, kvc)

    i32 = jnp.int32
    x1, s0, s1, g0i, g1i, r0, r1, cnt = pl.pallas_call(
        _oproj_route_body,
        grid=(N // MB,),
        in_specs=[
            pl.BlockSpec((MB, D), lambda i: (i, 0)),
            pl.BlockSpec((D, D), lambda i: (0, 0)),
            pl.BlockSpec((1, D), lambda i: (0, 0)),
            pl.BlockSpec((MB, D), lambda i: (i, 0)),
            pl.BlockSpec((1, D), lambda i: (0, 0)),
            pl.BlockSpec((D, E), lambda i: (0, 0)),
        ],
        out_specs=[
            pl.BlockSpec((MB, D), lambda i: (i, 0)),
            pl.BlockSpec((MB, 1), lambda i: (i, 0)),
            pl.BlockSpec((MB, 1), lambda i: (i, 0)),
            pl.BlockSpec((MB, 1), lambda i: (i, 0)),
            pl.BlockSpec((MB, 1), lambda i: (i, 0)),
            pl.BlockSpec((MB, 1), lambda i: (i, 0)),
            pl.BlockSpec((MB, 1), lambda i: (i, 0)),
            pl.BlockSpec((1, E), lambda i: (0, 0)),
        ],
        out_shape=[
            jax.ShapeDtypeStruct((N, D), f32),
            jax.ShapeDtypeStruct((N, 1), i32),
            jax.ShapeDtypeStruct((N, 1), i32),
            jax.ShapeDtypeStruct((N, 1), i32),
            jax.ShapeDtypeStruct((N, 1), i32),
            jax.ShapeDtypeStruct((N, 1), f32),
            jax.ShapeDtypeStruct((N, 1), f32),
            jax.ShapeDtypeStruct((1, E), i32),
        ],
        scratch_shapes=[pltpu.VMEM((1, E), f32)],
    )(attn, o_w.T, o_b.reshape(1, D), xf, norm1_w.reshape(1, D), gate_w.T)

    sidx = jnp.concatenate([s0, s1], axis=0).reshape(N * K // _CH, _CH)
    gidx = jnp.concatenate([g0i, g1i], axis=0).reshape(N * K // _CH, _CH)

    mesh = plsc.VectorSubcoreMesh(core_axis_name="c", subcore_axis_name="s")

    padded = pl.kernel(
        _disp_body,
        out_type=jax.ShapeDtypeStruct((NPAD, D), f32),
        mesh=mesh,
        scratch_types=[
            pltpu.VMEM((_RPW, _CH), i32),
            pltpu.VMEM((_CH, D), f32),
            pltpu.VMEM((_CH, D), f32),
            pltpu.SemaphoreType.DMA,
            pltpu.SemaphoreType.DMA,
            pltpu.SemaphoreType.DMA,
            pltpu.SemaphoreType.DMA,
        ],
    )(x1, sidx)

    outp = pl.pallas_call(
        _ffn_body,
        grid=(E,),
        in_specs=[
            pl.BlockSpec(memory_space=pltpu.SMEM),
            pl.BlockSpec((CAP, D), lambda e: (e, 0)),
            pl.BlockSpec((1, D, DFF), lambda e: (e, 0, 0)),
            pl.BlockSpec((1, DFF, D), lambda e: (e, 0, 0)),
        ],
        out_specs=pl.BlockSpec((CAP, D), lambda e: (e, 0)),
        out_shape=jax.ShapeDtypeStruct((NSLOT, D), f32),
    )(cnt, padded, w1, w2)

    gout = pl.kernel(
        _gath_body,
        out_type=jax.ShapeDtypeStruct((N * K, D), f32),
        mesh=mesh,
        scratch_types=[
            pltpu.VMEM((_RPW, _CH), i32),
            pltpu.VMEM((_CH, D), f32),
            pltpu.VMEM((_CH, D), f32),
            pltpu.SemaphoreType.DMA,
            pltpu.SemaphoreType.DMA,
            pltpu.SemaphoreType.DMA,
            pltpu.SemaphoreType.DMA,
        ],
    )(outp, gidx)

    out = pl.pallas_call(
        _combine_body,
        grid=(N // MB,),
        in_specs=[
            pl.BlockSpec((MB, D), lambda i: (i, 0)),
            pl.BlockSpec((MB, D), lambda i: (i, 0)),
            pl.BlockSpec((MB, D), lambda i: (i + N // MB, 0)),
            pl.BlockSpec((MB, 1), lambda i: (i, 0)),
            pl.BlockSpec((MB, 1), lambda i: (i, 0)),
            pl.BlockSpec((1, D), lambda i: (0, 0)),
        ],
        out_specs=pl.BlockSpec((MB, D), lambda i: (i, 0)),
        out_shape=jax.ShapeDtypeStruct((N, D), f32),
    )(x1, gout, gout, r0, r1, norm2_w.reshape(1, D))

    return out.reshape(B, S, D)


# speedup vs baseline: 1.8429x; 1.0000x over previous
import functools
import jax
import jax.numpy as jnp
from jax import lax
from jax.experimental import pallas as pl
from jax.experimental.pallas import tpu as pltpu
from jax.experimental.pallas import tpu_sc as plsc

B = 2
S = 2048
D = 1024
H = 16
KVH = 4
DH = D // H
REP = H // KVH
DFF = 2048
E = 64
K = 2
N = B * S
CAP = int(1.25 * N * K / E)
NSLOT = E * CAP
NPAD = NSLOT + 8
QB = 512
AQB = 256
MB = 256
SCALE = 1.0 / (DH ** 0.5)


def _proj_body(x_ref, w_ref, b_ref, q_ref, kv_ref):
    y = (
        jnp.dot(x_ref[...], w_ref[...], preferred_element_type=jnp.float32)
        + b_ref[...]
    )
    q_ref[...] = y[:, :D] * SCALE
    kv_ref[...] = y[:, D:]


def _attn_body(q_ref, kv_ref, o_ref):
    kv = kv_ref[...]
    for h in range(H):
        q = q_ref[:, h * DH:(h + 1) * DH]
        kh = h // REP
        k = kv[:, kh * DH:(kh + 1) * DH]
        v = kv[:, KVH * DH + kh * DH:KVH * DH + (kh + 1) * DH]
        s = lax.dot_general(q, k, (((1,), (1,)), ((), ())),
                            preferred_element_type=jnp.float32)
        m = jnp.max(s, axis=1, keepdims=True)
        p = jnp.exp(s - m)
        l = jnp.sum(p, axis=1, keepdims=True)
        o = jnp.dot(p, v, preferred_element_type=jnp.float32)
        o_ref[:, h * DH:(h + 1) * DH] = o / l


def _oproj_route_body(a_ref, w_ref, b_ref, src_ref, n1_ref, gw_ref,
                      x1_ref, s0_ref, s1_ref, g0_ref, g1_ref, r0_ref, r1_ref,
                      cnt_ref, carry):
    y = (
        jnp.dot(a_ref[...], w_ref[...], preferred_element_type=jnp.float32)
        + b_ref[...]
        + src_ref[...]
    )
    var = jnp.mean(y * y, axis=1, keepdims=True)
    x1 = n1_ref[...] * (y * lax.rsqrt(var + 1e-6))
    x1_ref[...] = x1
    probs = jax.nn.sigmoid(
        jnp.dot(x1, gw_ref[...], preferred_element_type=jnp.float32)
    )

    pid = pl.program_id(0)

    @pl.when(pid == 0)
    def _():
        carry[...] = jnp.zeros_like(carry)

    lanes = lax.broadcasted_iota(jnp.int32, (MB, E), 1)
    m1 = jnp.max(probs, axis=1, keepdims=True)
    a1 = jnp.min(jnp.where(probs == m1, lanes, E), axis=1, keepdims=True)
    pm = jnp.where(lanes == a1, -jnp.inf, probs)
    m2 = jnp.max(pm, axis=1, keepdims=True)
    a2 = jnp.min(jnp.where(pm == m2, lanes, E), axis=1, keepdims=True)
    oh0 = (lanes == a1).astype(jnp.float32)
    oh1 = (lanes == a2).astype(jnp.float32)
    ohs = oh0 + oh1
    ii = lax.broadcasted_iota(jnp.int32, (MB, MB), 0)
    jj = lax.broadcasted_iota(jnp.int32, (MB, MB), 1)
    tri = (jj < ii).astype(jnp.float32)
    cumex = (
        jnp.dot(tri, ohs, preferred_element_type=jnp.float32) + carry[...]
    )
    rank0 = jnp.sum(cumex * oh0, axis=1, keepdims=True)
    rank1 = jnp.sum(cumex * oh1, axis=1, keepdims=True)
    carry[...] = carry[...] + jnp.sum(ohs, axis=0, keepdims=True)
    v0 = rank0 < CAP
    v1 = rank1 < CAP
    sid0 = a1 * CAP + rank0.astype(jnp.int32)
    sid1 = a2 * CAP + rank1.astype(jnp.int32)
    s0_ref[...] = jnp.where(v0, sid0, NSLOT)
    s1_ref[...] = jnp.where(v1, sid1, NSLOT)
    g0_ref[...] = jnp.where(v0, sid0, 0)
    g1_ref[...] = jnp.where(v1, sid1, 0)
    ssum = m1 + m2 + 1e-6
    r0_ref[...] = jnp.where(v0, m1 / ssum, 0.0)
    r1_ref[...] = jnp.where(v1, m2 / ssum, 0.0)
    cnt_ref[...] = jnp.minimum(carry[...], float(CAP)).astype(jnp.int32)


def _ffn_body(cnt_ref, x_ref, w1_ref, w2_ref, o_ref):
    e = pl.program_id(0)
    c = pl.program_id(1)
    cap = cnt_ref[0, e]
    rows = lax.broadcasted_iota(jnp.int32, (CAP, 1), 0)
    x = jnp.where(rows < cap, x_ref[...], 0.0)
    h = jnp.dot(x, w1_ref[0], preferred_element_type=jnp.float32)
    h = 0.5 * h * (1.0 + lax.erf(h * (2.0 ** -0.5)))
    part = jnp.dot(h, w2_ref[0], preferred_element_type=jnp.float32)

    @pl.when(c == 0)
    def _():
        o_ref[...] = part

    @pl.when(c != 0)
    def _():
        o_ref[...] += part


def _combine_body(x1_ref, g0_ref, g1_ref, r0_ref, r1_ref, n2_ref, o_ref):
    y = x1_ref[...] + r0_ref[...] * g0_ref[...] + r1_ref[...] * g1_ref[...]
    var = jnp.mean(y * y, axis=1, keepdims=True)
    o_ref[...] = n2_ref[...] * (y * lax.rsqrt(var + 1e-6))


_RPW = 8
_CH = 32


def _disp_body(xf_hbm, sidx_hbm, padded_hbm, idx_v, rows0, rows1,
               lsem0, lsem1, ssem0, ssem1):
    info = plsc.get_sparse_core_info()
    nc = info.num_cores
    wid = lax.axis_index("s") * nc + lax.axis_index("c")
    base = wid * _RPW
    pltpu.sync_copy(sidx_hbm.at[pl.ds(base, _RPW)], idx_v)
    bufs = (rows0, rows1)
    lsems = (lsem0, lsem1)
    ssems = (ssem0, ssem1)

    def tok(jl):
        return lax.rem(base + jl, 128) * _CH

    loads = {}
    scats = {}
    loads[0] = pltpu.async_copy(xf_hbm.at[pl.ds(tok(0), _CH)], bufs[0],
                                lsems[0])
    for jl in range(_RPW):
        b = jl % 2
        if jl + 1 < _RPW:
            if jl - 1 >= 0:
                scats[jl - 1].wait()
            nb = (jl + 1) % 2
            loads[jl + 1] = pltpu.async_copy(
                xf_hbm.at[pl.ds(tok(jl + 1), _CH)], bufs[nb], lsems[nb])
        loads[jl].wait()
        scats[jl] = pltpu.async_copy(bufs[b], padded_hbm.at[idx_v.at[jl]],
                                     ssems[b])
    scats[_RPW - 2].wait()
    scats[_RPW - 1].wait()


def _gath_body(outp_hbm, gidx_hbm, gout_hbm, idx_v, rows0, rows1,
               lsem0, lsem1, ssem0, ssem1):
    info = plsc.get_sparse_core_info()
    nc = info.num_cores
    wid = lax.axis_index("s") * nc + lax.axis_index("c")
    base = wid * _RPW
    pltpu.sync_copy(gidx_hbm.at[pl.ds(base, _RPW)], idx_v)
    bufs = (rows0, rows1)
    lsems = (lsem0, lsem1)
    ssems = (ssem0, ssem1)
    loads = {}
    stores = {}
    loads[0] = pltpu.async_copy(outp_hbm.at[idx_v.at[0]], bufs[0], lsems[0])
    for jl in range(_RPW):
        b = jl % 2
        if jl + 1 < _RPW:
            if jl - 1 >= 0:
                stores[jl - 1].wait()
            nb = (jl + 1) % 2
            loads[jl + 1] = pltpu.async_copy(outp_hbm.at[idx_v.at[jl + 1]],
                                             bufs[nb], lsems[nb])
        loads[jl].wait()
        stores[jl] = pltpu.async_copy(
            bufs[b], gout_hbm.at[pl.ds((base + jl) * _CH, _CH)], ssems[b])
    stores[_RPW - 2].wait()
    stores[_RPW - 1].wait()


@jax.jit
def kernel(src, q_w, q_b, k_w, k_b, v_w, v_b, o_w, o_b, gate_w, w1, w2,
           norm1_w, norm2_w):
    f32 = jnp.float32
    xf = src.reshape(N, D)

    wcat = jnp.concatenate([q_w, k_w, v_w], axis=0).T
    bcat = jnp.concatenate([q_b, k_b, v_b]).reshape(1, -1)
    q, kvc = pl.pallas_call(
        _proj_body,
        grid=(N // QB,),
        in_specs=[
            pl.BlockSpec((QB, D), lambda i: (i, 0)),
            pl.BlockSpec((D, D + 2 * KVH * DH), lambda i: (0, 0)),
            pl.BlockSpec((1, D + 2 * KVH * DH), lambda i: (0, 0)),
        ],
        out_specs=[
            pl.BlockSpec((QB, D), lambda i: (i, 0)),
            pl.BlockSpec((QB, 2 * KVH * DH), lambda i: (i, 0)),
        ],
        out_shape=[
            jax.ShapeDtypeStruct((N, D), f32),
            jax.ShapeDtypeStruct((N, 2 * KVH * DH), f32),
        ],
    )(xf, wcat, bcat)

    attn = pl.pallas_call(
        _attn_body,
        grid=(B, S // AQB),
        in_specs=[
            pl.BlockSpec((AQB, D), lambda b, i: (b * (S // AQB) + i, 0)),
            pl.BlockSpec((S, 2 * KVH * DH), lambda b, i: (b, 0)),
        ],
        out_specs=pl.BlockSpec((AQB, D), lambda b, i: (b * (S // AQB) + i, 0)),
        out_shape=jax.ShapeDtypeStruct((N, D), f32),
    )(q, kvc)

    i32 = jnp.int32
    x1, s0, s1, g0i, g1i, r0, r1, cnt = pl.pallas_call(
        _oproj_route_body,
        grid=(N // MB,),
        in_specs=[
            pl.BlockSpec((MB, D), lambda i: (i, 0)),
            pl.BlockSpec((D, D), lambda i: (0, 0)),
            pl.BlockSpec((1, D), lambda i: (0, 0)),
            pl.BlockSpec((MB, D), lambda i: (i, 0)),
            pl.BlockSpec((1, D), lambda i: (0, 0)),
            pl.BlockSpec((D, E), lambda i: (0, 0)),
        ],
        out_specs=[
            pl.BlockSpec((MB, D), lambda i: (i, 0)),
            pl.BlockSpec((MB, 1), lambda i: (i, 0)),
            pl.BlockSpec((MB, 1), lambda i: (i, 0)),
            pl.BlockSpec((MB, 1), lambda i: (i, 0)),
            pl.BlockSpec((MB, 1), lambda i: (i, 0)),
            pl.BlockSpec((MB, 1), lambda i: (i, 0)),
            pl.BlockSpec((MB, 1), lambda i: (i, 0)),
            pl.BlockSpec((1, E), lambda i: (0, 0)),
        ],
        out_shape=[
            jax.ShapeDtypeStruct((N, D), f32),
            jax.ShapeDtypeStruct((N, 1), i32),
            jax.ShapeDtypeStruct((N, 1), i32),
            jax.ShapeDtypeStruct((N, 1), i32),
            jax.ShapeDtypeStruct((N, 1), i32),
            jax.ShapeDtypeStruct((N, 1), f32),
            jax.ShapeDtypeStruct((N, 1), f32),
            jax.ShapeDtypeStruct((1, E), i32),
        ],
        scratch_shapes=[pltpu.VMEM((1, E), f32)],
    )(attn, o_w.T, o_b.reshape(1, D), xf, norm1_w.reshape(1, D), gate_w.T)

    sidx = jnp.concatenate([s0, s1], axis=0).reshape(N * K // _CH, _CH)
    gidx = jnp.concatenate([g0i, g1i], axis=0).reshape(N * K // _CH, _CH)

    mesh = plsc.VectorSubcoreMesh(core_axis_name="c", subcore_axis_name="s")

    padded = pl.kernel(
        _disp_body,
        out_type=jax.ShapeDtypeStruct((NPAD, D), f32),
        mesh=mesh,
        scratch_types=[
            pltpu.VMEM((_RPW, _CH), i32),
            pltpu.VMEM((_CH, D), f32),
            pltpu.VMEM((_CH, D), f32),
            pltpu.SemaphoreType.DMA,
            pltpu.SemaphoreType.DMA,
            pltpu.SemaphoreType.DMA,
            pltpu.SemaphoreType.DMA,
        ],
    )(x1, sidx)

    DC = 2
    outp = pl.pallas_call(
        _ffn_body,
        grid=(E, DC),
        in_specs=[
            pl.BlockSpec(memory_space=pltpu.SMEM),
            pl.BlockSpec((CAP, D), lambda e, c: (e, 0)),
            pl.BlockSpec((1, D, DFF // DC), lambda e, c: (e, 0, c)),
            pl.BlockSpec((1, DFF // DC, D), lambda e, c: (e, c, 0)),
        ],
        out_specs=pl.BlockSpec((CAP, D), lambda e, c: (e, 0)),
        out_shape=jax.ShapeDtypeStruct((NSLOT, D), f32),
    )(cnt, padded, w1, w2)

    gout = pl.kernel(
        _gath_body,
        out_type=jax.ShapeDtypeStruct((N * K, D), f32),
        mesh=mesh,
        scratch_types=[
            pltpu.VMEM((_RPW, _CH), i32),
            pltpu.VMEM((_CH, D), f32),
            pltpu.VMEM((_CH, D), f32),
            pltpu.SemaphoreType.DMA,
            pltpu.SemaphoreType.DMA,
            pltpu.SemaphoreType.DMA,
            pltpu.SemaphoreType.DMA,
        ],
    )(outp, gidx)

    out = pl.pallas_call(
        _combine_body,
        grid=(N // MB,),
        in_specs=[
            pl.BlockSpec((MB, D), lambda i: (i, 0)),
            pl.BlockSpec((MB, D), lambda i: (i, 0)),
            pl.BlockSpec((MB, D), lambda i: (i + N // MB, 0)),
            pl.BlockSpec((MB, 1), lambda i: (i, 0)),
            pl.BlockSpec((MB, 1), lambda i: (i, 0)),
            pl.BlockSpec((1, D), lambda i: (0, 0)),
        ],
        out_specs=pl.BlockSpec((MB, D), lambda i: (i, 0)),
        out_shape=jax.ShapeDtypeStruct((N, D), f32),
    )(x1, gout, gout, r0, r1, norm2_w.reshape(1, D))

    return out.reshape(B, S, D)


# speedup vs baseline: 1.8446x; 1.0009x over previous
import functools
import jax
import jax.numpy as jnp
from jax import lax
from jax.experimental import pallas as pl
from jax.experimental.pallas import tpu as pltpu
from jax.experimental.pallas import tpu_sc as plsc

B = 2
S = 2048
D = 1024
H = 16
KVH = 4
DH = D // H
REP = H // KVH
DFF = 2048
E = 64
K = 2
N = B * S
CAP = int(1.25 * N * K / E)
NSLOT = E * CAP
NPAD = NSLOT + 8
QB = 512
AQB = 256
MB = 256
SCALE = 1.0 / (DH ** 0.5)


def _proj_body(x_ref, w_ref, b_ref, q_ref, kv_ref):
    y = (
        jnp.dot(x_ref[...], w_ref[...], preferred_element_type=jnp.float32)
        + b_ref[...]
    )
    q_ref[...] = y[:, :D] * SCALE
    kv_ref[...] = y[:, D:]


def _attn_body(q_ref, kv_ref, o_ref):
    kv = kv_ref[...]
    for h in range(H):
        q = q_ref[:, h * DH:(h + 1) * DH]
        kh = h // REP
        k = kv[:, kh * DH:(kh + 1) * DH]
        v = kv[:, KVH * DH + kh * DH:KVH * DH + (kh + 1) * DH]
        s = lax.dot_general(q, k, (((1,), (1,)), ((), ())),
                            preferred_element_type=jnp.float32)
        m = jnp.max(s, axis=1, keepdims=True)
        p = jnp.exp(s - m)
        l = jnp.sum(p, axis=1, keepdims=True)
        o = jnp.dot(p, v, preferred_element_type=jnp.float32)
        o_ref[:, h * DH:(h + 1) * DH] = o / l


def _oproj_route_body(a_ref, w_ref, b_ref, src_ref, n1_ref, gw_ref,
                      x1_ref, s0_ref, s1_ref, g0_ref, g1_ref, r0_ref, r1_ref,
                      cnt_ref, carry):
    y = (
        jnp.dot(a_ref[...], w_ref[...], preferred_element_type=jnp.float32)
        + b_ref[...]
        + src_ref[...]
    )
    var = jnp.mean(y * y, axis=1, keepdims=True)
    x1 = n1_ref[...] * (y * lax.rsqrt(var + 1e-6))
    x1_ref[...] = x1
    probs = jax.nn.sigmoid(
        jnp.dot(x1, gw_ref[...], preferred_element_type=jnp.float32)
    )

    pid = pl.program_id(0)

    @pl.when(pid == 0)
    def _():
        carry[...] = jnp.zeros_like(carry)

    lanes = lax.broadcasted_iota(jnp.int32, (MB, E), 1)
    m1 = jnp.max(probs, axis=1, keepdims=True)
    a1 = jnp.min(jnp.where(probs == m1, lanes, E), axis=1, keepdims=True)
    pm = jnp.where(lanes == a1, -jnp.inf, probs)
    m2 = jnp.max(pm, axis=1, keepdims=True)
    a2 = jnp.min(jnp.where(pm == m2, lanes, E), axis=1, keepdims=True)
    oh0 = (lanes == a1).astype(jnp.float32)
    oh1 = (lanes == a2).astype(jnp.float32)
    ohs = oh0 + oh1
    ii = lax.broadcasted_iota(jnp.int32, (MB, MB), 0)
    jj = lax.broadcasted_iota(jnp.int32, (MB, MB), 1)
    tri = (jj < ii).astype(jnp.float32)
    cumex = (
        jnp.dot(tri, ohs, preferred_element_type=jnp.float32) + carry[...]
    )
    rank0 = jnp.sum(cumex * oh0, axis=1, keepdims=True)
    rank1 = jnp.sum(cumex * oh1, axis=1, keepdims=True)
    carry[...] = carry[...] + jnp.sum(ohs, axis=0, keepdims=True)
    v0 = rank0 < CAP
    v1 = rank1 < CAP
    sid0 = a1 * CAP + rank0.astype(jnp.int32)
    sid1 = a2 * CAP + rank1.astype(jnp.int32)
    s0_ref[...] = jnp.where(v0, sid0, NSLOT)
    s1_ref[...] = jnp.where(v1, sid1, NSLOT)
    g0_ref[...] = jnp.where(v0, sid0, 0)
    g1_ref[...] = jnp.where(v1, sid1, 0)
    ssum = m1 + m2 + 1e-6
    r0_ref[...] = jnp.where(v0, m1 / ssum, 0.0)
    r1_ref[...] = jnp.where(v1, m2 / ssum, 0.0)
    cnt_ref[...] = jnp.minimum(carry[...], float(CAP)).astype(jnp.int32)


def _ffn_body(cnt_ref, x_ref, w1_ref, w2_ref, o_ref):
    e = pl.program_id(0)
    cap = cnt_ref[0, e]
    rows = lax.broadcasted_iota(jnp.int32, (CAP, 1), 0)
    x = jnp.where(rows < cap, x_ref[...], 0.0)
    h = jnp.dot(x, w1_ref[0], preferred_element_type=jnp.float32)
    h = 0.5 * h * (1.0 + lax.erf(h * (2.0 ** -0.5)))
    o_ref[...] = jnp.dot(h, w2_ref[0], preferred_element_type=jnp.float32)


def _combine_body(x1_ref, g0_ref, g1_ref, r0_ref, r1_ref, n2_ref, o_ref):
    y = x1_ref[...] + r0_ref[...] * g0_ref[...] + r1_ref[...] * g1_ref[...]
    var = jnp.mean(y * y, axis=1, keepdims=True)
    o_ref[...] = n2_ref[...] * (y * lax.rsqrt(var + 1e-6))


_RPW = 8
_CH = 32


def _disp_body(xf_hbm, sidx_hbm, padded_hbm, idx_v, rows0, rows1,
               lsem0, lsem1, ssem0, ssem1):
    info = plsc.get_sparse_core_info()
    nc = info.num_cores
    wid = lax.axis_index("s") * nc + lax.axis_index("c")
    base = wid * _RPW
    pltpu.sync_copy(sidx_hbm.at[pl.ds(base, _RPW)], idx_v)
    bufs = (rows0, rows1)
    lsems = (lsem0, lsem1)
    ssems = (ssem0, ssem1)

    def tok(jl):
        return lax.rem(base + jl, 128) * _CH

    loads = {}
    scats = {}
    loads[0] = pltpu.async_copy(xf_hbm.at[pl.ds(tok(0), _CH)], bufs[0],
                                lsems[0])
    for jl in range(_RPW):
        b = jl % 2
        if jl + 1 < _RPW:
            if jl - 1 >= 0:
                scats[jl - 1].wait()
            nb = (jl + 1) % 2
            loads[jl + 1] = pltpu.async_copy(
                xf_hbm.at[pl.ds(tok(jl + 1), _CH)], bufs[nb], lsems[nb])
        loads[jl].wait()
        scats[jl] = pltpu.async_copy(bufs[b], padded_hbm.at[idx_v.at[jl]],
                                     ssems[b])
    scats[_RPW - 2].wait()
    scats[_RPW - 1].wait()


def _gath_body(outp_hbm, gidx_hbm, gout_hbm, idx_v, rows0, rows1,
               lsem0, lsem1, ssem0, ssem1):
    info = plsc.get_sparse_core_info()
    nc = info.num_cores
    wid = lax.axis_index("s") * nc + lax.axis_index("c")
    base = wid * _RPW
    pltpu.sync_copy(gidx_hbm.at[pl.ds(base, _RPW)], idx_v)
    bufs = (rows0, rows1)
    lsems = (lsem0, lsem1)
    ssems = (ssem0, ssem1)
    loads = {}
    stores = {}
    loads[0] = pltpu.async_copy(outp_hbm.at[idx_v.at[0]], bufs[0], lsems[0])
    for jl in range(_RPW):
        b = jl % 2
        if jl + 1 < _RPW:
            if jl - 1 >= 0:
                stores[jl - 1].wait()
            nb = (jl + 1) % 2
            loads[jl + 1] = pltpu.async_copy(outp_hbm.at[idx_v.at[jl + 1]],
                                             bufs[nb], lsems[nb])
        loads[jl].wait()
        stores[jl] = pltpu.async_copy(
            bufs[b], gout_hbm.at[pl.ds((base + jl) * _CH, _CH)], ssems[b])
    stores[_RPW - 2].wait()
    stores[_RPW - 1].wait()


@jax.jit
def kernel(src, q_w, q_b, k_w, k_b, v_w, v_b, o_w, o_b, gate_w, w1, w2,
           norm1_w, norm2_w):
    f32 = jnp.float32
    xf = src.reshape(N, D)

    wcat = jnp.concatenate([q_w, k_w, v_w], axis=0).T
    bcat = jnp.concatenate([q_b, k_b, v_b]).reshape(1, -1)
    q, kvc = pl.pallas_call(
        _proj_body,
        grid=(N // QB,),
        in_specs=[
            pl.BlockSpec((QB, D), lambda i: (i, 0)),
            pl.BlockSpec((D, D + 2 * KVH * DH), lambda i: (0, 0)),
            pl.BlockSpec((1, D + 2 * KVH * DH), lambda i: (0, 0)),
        ],
        out_specs=[
            pl.BlockSpec((QB, D), lambda i: (i, 0)),
            pl.BlockSpec((QB, 2 * KVH * DH), lambda i: (i, 0)),
        ],
        out_shape=[
            jax.ShapeDtypeStruct((N, D), f32),
            jax.ShapeDtypeStruct((N, 2 * KVH * DH), f32),
        ],
    )(xf, wcat, bcat)

    attn = pl.pallas_call(
        _attn_body,
        grid=(B, S // AQB),
        in_specs=[
            pl.BlockSpec((AQB, D), lambda b, i: (b * (S // AQB) + i, 0)),
            pl.BlockSpec((S, 2 * KVH * DH), lambda b, i: (b, 0)),
        ],
        out_specs=pl.BlockSpec((AQB, D), lambda b, i: (b * (S // AQB) + i, 0)),
        out_shape=jax.ShapeDtypeStruct((N, D), f32),
    )(q, kvc)

    i32 = jnp.int32
    x1, s0, s1, g0i, g1i, r0, r1, cnt = pl.pallas_call(
        _oproj_route_body,
        grid=(N // MB,),
        in_specs=[
            pl.BlockSpec((MB, D), lambda i: (i, 0)),
            pl.BlockSpec((D, D), lambda i: (0, 0)),
            pl.BlockSpec((1, D), lambda i: (0, 0)),
            pl.BlockSpec((MB, D), lambda i: (i, 0)),
            pl.BlockSpec((1, D), lambda i: (0, 0)),
            pl.BlockSpec((D, E), lambda i: (0, 0)),
        ],
        out_specs=[
            pl.BlockSpec((MB, D), lambda i: (i, 0)),
            pl.BlockSpec((MB, 1), lambda i: (i, 0)),
            pl.BlockSpec((MB, 1), lambda i: (i, 0)),
            pl.BlockSpec((MB, 1), lambda i: (i, 0)),
            pl.BlockSpec((MB, 1), lambda i: (i, 0)),
            pl.BlockSpec((MB, 1), lambda i: (i, 0)),
            pl.BlockSpec((MB, 1), lambda i: (i, 0)),
            pl.BlockSpec((1, E), lambda i: (0, 0)),
        ],
        out_shape=[
            jax.ShapeDtypeStruct((N, D), f32),
            jax.ShapeDtypeStruct((N, 1), i32),
            jax.ShapeDtypeStruct((N, 1), i32),
            jax.ShapeDtypeStruct((N, 1), i32),
            jax.ShapeDtypeStruct((N, 1), i32),
            jax.ShapeDtypeStruct((N, 1), f32),
            jax.ShapeDtypeStruct((N, 1), f32),
            jax.ShapeDtypeStruct((1, E), i32),
        ],
        scratch_shapes=[pltpu.VMEM((1, E), f32)],
    )(attn, o_w.T, o_b.reshape(1, D), xf, norm1_w.reshape(1, D), gate_w.T)

    sidx = jnp.concatenate([s0, s1], axis=0).reshape(N * K // _CH, _CH)
    gidx = jnp.concatenate([g0i, g1i], axis=0).reshape(N * K // _CH, _CH)

    mesh = plsc.VectorSubcoreMesh(core_axis_name="c", subcore_axis_name="s")

    padded = pl.kernel(
        _disp_body,
        out_type=jax.ShapeDtypeStruct((NPAD, D), f32),
        mesh=mesh,
        scratch_types=[
            pltpu.VMEM((_RPW, _CH), i32),
            pltpu.VMEM((_CH, D), f32),
            pltpu.VMEM((_CH, D), f32),
            pltpu.SemaphoreType.DMA,
            pltpu.SemaphoreType.DMA,
            pltpu.SemaphoreType.DMA,
            pltpu.SemaphoreType.DMA,
        ],
    )(x1, sidx)

    outp = pl.pallas_call(
        _ffn_body,
        grid=(E,),
        in_specs=[
            pl.BlockSpec(memory_space=pltpu.SMEM),
            pl.BlockSpec((CAP, D), lambda e: (e, 0)),
            pl.BlockSpec((1, D, DFF), lambda e: (e, 0, 0)),
            pl.BlockSpec((1, DFF, D), lambda e: (e, 0, 0)),
        ],
        out_specs=pl.BlockSpec((CAP, D), lambda e: (e, 0)),
        out_shape=jax.ShapeDtypeStruct((NSLOT, D), f32),
    )(cnt, padded, w1, w2)

    gout = pl.kernel(
        _gath_body,
        out_type=jax.ShapeDtypeStruct((N * K, D), f32),
        mesh=mesh,
        scratch_types=[
            pltpu.VMEM((_RPW, _CH), i32),
            pltpu.VMEM((_CH, D), f32),
            pltpu.VMEM((_CH, D), f32),
            pltpu.SemaphoreType.DMA,
            pltpu.SemaphoreType.DMA,
            pltpu.SemaphoreType.DMA,
            pltpu.SemaphoreType.DMA,
        ],
    )(outp, gidx)

    out = pl.pallas_call(
        _combine_body,
        grid=(N // MB,),
        in_specs=[
            pl.BlockSpec((MB, D), lambda i: (i, 0)),
            pl.BlockSpec((MB, D), lambda i: (i, 0)),
            pl.BlockSpec((MB, D), lambda i: (i + N // MB, 0)),
            pl.BlockSpec((MB, 1), lambda i: (i, 0)),
            pl.BlockSpec((MB, 1), lambda i: (i, 0)),
            pl.BlockSpec((1, D), lambda i: (0, 0)),
        ],
        out_specs=pl.BlockSpec((MB, D), lambda i: (i, 0)),
        out_shape=jax.ShapeDtypeStruct((N, D), f32),
    )(x1, gout, gout, r0, r1, norm2_w.reshape(1, D))

    return out.reshape(B, S, D)
